# Initial kernel scaffold; baseline (speedup 1.0000x reference)
import functools

import jax
import jax.numpy as jnp
from jax import lax
from jax.experimental import pallas as pl
from jax.experimental.pallas import tpu as pltpu
from jax.experimental.pallas import tpu_sc as plsc

N_VARS = 1_000_000
E = 8_000_000
OUT = 2_000_000
FAN = 4

NC = 2
NS = 16
NW = NC * NS
L = 16

SHIFT = 6
XS_SIZE = 2_000_024

BC = 6_256
NBC = 10
W_PAD = 1_008_000

CH = 8_000
CE = CH * FAN
NCHUNK = OUT // CH


def _worker_id():
    return lax.axis_index("s") * NC + lax.axis_index("c")


def _gather_reduce(table, p_hbm, out_hbm, idxbuf, gbuf, obuf, sem, iota,
                   shift, is_prod):
    wid = _worker_id()
    nk = (NCHUNK - 1 - wid) // NW + 1

    def chunk(k, _):
        c = wid + k * NW
        pltpu.sync_copy(p_hbm.at[pl.ds(c * CE, CE)], idxbuf)
        if shift:
            sft = jnp.full((L,), shift, jnp.int32)

            def shift8(v, _):
                for u in range(8):
                    plsc.addupdate(idxbuf.at[pl.ds((v * 8 + u) * L, L)], sft)
                return 0

            lax.fori_loop(0, CE // L // 8, shift8, 0)
        pltpu.async_copy(table.at[idxbuf], gbuf, sem).wait()

        def red_vreg(q, _):
            b = iota * FAN + q * (FAN * L)
            a0 = plsc.load_gather(gbuf, [b])
            a1 = plsc.load_gather(gbuf, [b + 1])
            a2 = plsc.load_gather(gbuf, [b + 2])
            a3 = plsc.load_gather(gbuf, [b + 3])
            if is_prod:
                obuf[pl.ds(q * L, L)] = (a0 * a1) * (a2 * a3)
            else:
                obuf[pl.ds(q * L, L)] = (a0 + a1) + (a2 + a3)
            return 0

        lax.fori_loop(0, CH // L, red_vreg, 0)
        pltpu.sync_copy(obuf, out_hbm.at[pl.ds(c * CH, CH)])
        return 0

    lax.fori_loop(0, nk, chunk, 0)


def _layer0_body(w_hbm, p_hbm, y0_hbm, xs, wbuf, ibuf, idxbuf, gbuf, obuf,
                 cbuf, sem):
    sid = lax.axis_index("s")
    iota = lax.broadcasted_iota(jnp.int32, (L,), 0)

    start = (N_VARS // NS) * sid // 8 * 8

    def build_chunk(c, _):
        src = start + c * BC
        pltpu.sync_copy(w_hbm.at[pl.ds(src, BC)], wbuf)

        def build_vreg(v, _):
            wv = wbuf[pl.ds(v * L, L)]
            b2 = iota * 2 + v * (2 * L)
            plsc.store_scatter(ibuf, [b2], wv)
            plsc.store_scatter(ibuf, [b2 + 1], 1.0 - wv)
            return 0

        lax.fori_loop(0, BC // L, build_vreg, 0)
        pltpu.sync_copy(ibuf, xs.at[pl.ds(8 + 2 * src, 2 * BC)])
        return 0

    lax.fori_loop(0, NBC, build_chunk, 0)

    @pl.when(sid == 0)
    def _():
        cbuf[...] = jnp.where(iota == 7, 1.0, 0.0).astype(jnp.float32)
        pltpu.sync_copy(cbuf.at[pl.ds(0, 8)], xs.at[pl.ds(0, 8)])

    plsc.subcore_barrier()

    _gather_reduce(xs, p_hbm, y0_hbm, idxbuf, gbuf, obuf, sem, iota,
                   SHIFT, True)


def _layer1_body(y0_hbm, p_hbm, y1_hbm, ys, idxbuf, gbuf, obuf, sem):
    sid = lax.axis_index("s")
    iota = lax.broadcasted_iota(jnp.int32, (L,), 0)

    seg = OUT // NS
    pltpu.sync_copy(y0_hbm.at[pl.ds(sid * seg, seg)],
                    ys.at[pl.ds(sid * seg, seg)])
    plsc.subcore_barrier()

    _gather_reduce(ys, p_hbm, y1_hbm, idxbuf, gbuf, obuf, sem, iota,
                   0, False)


@functools.cache
def _build_calls():
    mesh = plsc.VectorSubcoreMesh(core_axis_name="c", subcore_axis_name="s")
    layer0 = pl.kernel(
        _layer0_body,
        out_type=jax.ShapeDtypeStruct((OUT,), jnp.float32),
        mesh=mesh,
        scratch_types=[
            pltpu.VMEM_SHARED((XS_SIZE,), jnp.float32),
            pltpu.VMEM((BC,), jnp.float32),
            pltpu.VMEM((2 * BC,), jnp.float32),
            pltpu.VMEM((CE,), jnp.int32),
            pltpu.VMEM((CE,), jnp.float32),
            pltpu.VMEM((CH,), jnp.float32),
            pltpu.VMEM((L,), jnp.float32),
            pltpu.SemaphoreType.DMA,
        ],
    )
    layer1 = pl.kernel(
        _layer1_body,
        out_type=jax.ShapeDtypeStruct((OUT,), jnp.float32),
        mesh=mesh,
        scratch_types=[
            pltpu.VMEM_SHARED((OUT,), jnp.float32),
            pltpu.VMEM((CE,), jnp.int32),
            pltpu.VMEM((CE,), jnp.float32),
            pltpu.VMEM((CH,), jnp.float32),
            pltpu.SemaphoreType.DMA,
        ],
    )
    return layer0, layer1


def kernel(weights, ptrs0, ptrs1):
    layer0, layer1 = _build_calls()
    w_pad = jnp.concatenate(
        [weights, jnp.zeros((W_PAD - N_VARS,), jnp.float32)])
    y0 = layer0(w_pad, ptrs0)
    y1 = layer1(y0, ptrs1)
    return y1


# speedup vs baseline: 307.9267x; 307.9267x over previous
import functools

import jax
import jax.numpy as jnp
from jax import lax
from jax.experimental import pallas as pl
from jax.experimental.pallas import tpu as pltpu
from jax.experimental.pallas import tpu_sc as plsc

N_VARS = 1_000_000
E = 8_000_000
OUT = 2_000_000
FAN = 4

NC = 2
NS = 16
NW = NC * NS
L = 16

SHIFT = 6
XS_SIZE = 2_000_128

BC = 15_632
NBC = 2
W_PAD = 1_000_008

CH = 8_000
CE = CH * FAN
NCHUNK = OUT // CH


def _worker_id():
    return lax.axis_index("s") * NC + lax.axis_index("c")


def _encode_body(w_hbm, xs_hbm, wbuf, ibuf, cbuf, sem):
    wid = _worker_id()
    iota = lax.broadcasted_iota(jnp.int32, (L,), 0)
    start = (N_VARS // NW) * wid // 8 * 8

    def build_chunk(c, _):
        src = start + c * BC
        pltpu.sync_copy(w_hbm.at[pl.ds(src, BC)], wbuf)

        def build_vreg(v, _):
            wv = wbuf[pl.ds(v * L, L)]
            b2 = iota * 2 + v * (2 * L)
            plsc.store_scatter(ibuf, [b2], wv)
            plsc.store_scatter(ibuf, [b2 + 1], 1.0 - wv)
            return 0

        lax.fori_loop(0, BC // L, build_vreg, 0)
        pltpu.sync_copy(ibuf, xs_hbm.at[pl.ds(8 + 2 * src, 2 * BC)])
        return 0

    lax.fori_loop(0, NBC, build_chunk, 0)

    @pl.when(wid == 0)
    def _():
        cbuf[...] = jnp.where(iota == 7, 1.0, 0.0).astype(jnp.float32)
        pltpu.sync_copy(cbuf.at[pl.ds(0, 8)], xs_hbm.at[pl.ds(0, 8)])


def _make_gather_body(shift, is_prod):
    def body(table_hbm, p_hbm, out_hbm, idxbuf, gbuf, obuf, sem):
        wid = _worker_id()
        iota = lax.broadcasted_iota(jnp.int32, (L,), 0)
        nk = (NCHUNK - 1 - wid) // NW + 1

        def chunk(k, _):
            c = wid + k * NW
            pltpu.sync_copy(p_hbm.at[pl.ds(c * CE, CE)], idxbuf)
            if shift:
                sft = jnp.full((L,), shift, jnp.int32)

                def shift8(v, _):
                    for u in range(8):
                        plsc.addupdate(
                            idxbuf.at[pl.ds((v * 8 + u) * L, L)], sft)
                    return 0

                lax.fori_loop(0, CE // L // 8, shift8, 0)
            pltpu.async_copy(table_hbm.at[idxbuf], gbuf, sem).wait()

            def red_vreg(q, _):
                b = iota * FAN + q * (FAN * L)
                a0 = plsc.load_gather(gbuf, [b])
                a1 = plsc.load_gather(gbuf, [b + 1])
                a2 = plsc.load_gather(gbuf, [b + 2])
                a3 = plsc.load_gather(gbuf, [b + 3])
                if is_prod:
                    obuf[pl.ds(q * L, L)] = (a0 * a1) * (a2 * a3)
                else:
                    obuf[pl.ds(q * L, L)] = (a0 + a1) + (a2 + a3)
                return 0

            lax.fori_loop(0, CH // L, red_vreg, 0)
            pltpu.sync_copy(obuf, out_hbm.at[pl.ds(c * CH, CH)])
            return 0

        lax.fori_loop(0, nk, chunk, 0)

    return body


@functools.cache
def _build_calls():
    mesh = plsc.VectorSubcoreMesh(core_axis_name="c", subcore_axis_name="s")
    params = pltpu.CompilerParams(needs_layout_passes=False)
    encode = pl.kernel(
        _encode_body,
        out_type=jax.ShapeDtypeStruct((XS_SIZE,), jnp.float32),
        mesh=mesh,
        compiler_params=params,
        scratch_types=[
            pltpu.VMEM((BC,), jnp.float32),
            pltpu.VMEM((2 * BC,), jnp.float32),
            pltpu.VMEM((L,), jnp.float32),
            pltpu.SemaphoreType.DMA,
        ],
    )
    gather_scratch = [
        pltpu.VMEM((CE,), jnp.int32),
        pltpu.VMEM((CE,), jnp.float32),
        pltpu.VMEM((CH,), jnp.float32),
        pltpu.SemaphoreType.DMA,
    ]
    layer0 = pl.kernel(
        _make_gather_body(SHIFT, True),
        out_type=jax.ShapeDtypeStruct((OUT,), jnp.float32),
        mesh=mesh,
        compiler_params=params,
        scratch_types=gather_scratch,
    )
    layer1 = pl.kernel(
        _make_gather_body(0, False),
        out_type=jax.ShapeDtypeStruct((OUT,), jnp.float32),
        mesh=mesh,
        compiler_params=params,
        scratch_types=gather_scratch,
    )
    return encode, layer0, layer1


def kernel(weights, ptrs0, ptrs1):
    encode, layer0, layer1 = _build_calls()
    w_pad = jnp.concatenate(
        [weights, jnp.zeros((W_PAD - N_VARS,), jnp.float32)])
    xs = encode(w_pad)
    y0 = layer0(xs, ptrs0)
    y1 = layer1(y0, ptrs1)
    return y1


# speedup vs baseline: 354.1114x; 1.1500x over previous
import functools

import jax
import jax.numpy as jnp
from jax import lax
from jax.experimental import pallas as pl
from jax.experimental.pallas import tpu as pltpu
from jax.experimental.pallas import tpu_sc as plsc

N_VARS = 1_000_000
E = 8_000_000
OUT = 2_000_000
FAN = 4

NC = 2
NS = 16
NW = NC * NS
L = 16

SHIFT = 6
XS_SIZE = 2_000_128

BC = 15_632
NBC = 2
W_PAD = 1_000_008

CH = 4_000
CE = CH * FAN
NCHUNK = OUT // CH
NKMAX = (NCHUNK - 1) // NW + 1


def _worker_id():
    return lax.axis_index("s") * NC + lax.axis_index("c")


def _encode_body(w_hbm, xs_hbm, wbuf, ibuf, cbuf, sem):
    wid = _worker_id()
    iota = lax.broadcasted_iota(jnp.int32, (L,), 0)
    start = (N_VARS // NW) * wid // 8 * 8

    def build_chunk(c, _):
        src = start + c * BC
        pltpu.sync_copy(w_hbm.at[pl.ds(src, BC)], wbuf)

        def build_vreg(v, _):
            wv = wbuf[pl.ds(v * L, L)]
            b2 = iota * 2 + v * (2 * L)
            plsc.store_scatter(ibuf, [b2], wv)
            plsc.store_scatter(ibuf, [b2 + 1], 1.0 - wv)
            return 0

        lax.fori_loop(0, BC // L, build_vreg, 0)
        pltpu.sync_copy(ibuf, xs_hbm.at[pl.ds(8 + 2 * src, 2 * BC)])
        return 0

    lax.fori_loop(0, NBC, build_chunk, 0)

    @pl.when(wid == 0)
    def _():
        cbuf[...] = jnp.where(iota == 7, 1.0, 0.0).astype(jnp.float32)
        pltpu.sync_copy(cbuf.at[pl.ds(0, 8)], xs_hbm.at[pl.ds(0, 8)])


def _make_gather_body(shift, is_prod):
    def body(table_hbm, p_hbm, out_hbm, idx0, idx1, g0, g1, o0, o1,
             si0, si1, sg0, sg1, so0, so1):
        wid = _worker_id()
        iota = lax.broadcasted_iota(jnp.int32, (L,), 0)
        idxb, gb, ob = (idx0, idx1), (g0, g1), (o0, o1)
        sib, sgb, sob = (si0, si1), (sg0, sg1), (so0, so1)

        def cid(j):
            return wid + j * NW

        def start_idx(j, b):
            pltpu.async_copy(p_hbm.at[pl.ds(cid(j) * CE, CE)], idxb[b],
                             sib[b])

        def shift_idx(b):
            if not shift:
                return
            sft = jnp.full((L,), shift, jnp.int32)

            def shift8(v, _):
                for u in range(8):
                    plsc.addupdate(idxb[b].at[pl.ds((v * 8 + u) * L, L)],
                                   sft)
                return 0

            lax.fori_loop(0, CE // L // 8, shift8, 0)

        def start_gather(b):
            pltpu.make_async_copy(p_hbm.at[pl.ds(0, CE)], idxb[b],
                                  sib[b]).wait()
            shift_idx(b)
            pltpu.async_copy(table_hbm.at[idxb[b]], gb[b], sgb[b])

        def reduce_store(j, b):
            def red_vreg(q, _):
                base = iota * FAN + q * (FAN * L)
                a0 = plsc.load_gather(gb[b], [base])
                a1 = plsc.load_gather(gb[b], [base + 1])
                a2 = plsc.load_gather(gb[b], [base + 2])
                a3 = plsc.load_gather(gb[b], [base + 3])
                if is_prod:
                    ob[b][pl.ds(q * L, L)] = (a0 * a1) * (a2 * a3)
                else:
                    ob[b][pl.ds(q * L, L)] = (a0 + a1) + (a2 + a3)
                return 0

            lax.fori_loop(0, CH // L, red_vreg, 0)
            pltpu.async_copy(ob[b], out_hbm.at[pl.ds(cid(j) * CH, CH)],
                             sob[b])

        start_idx(0, 0)
        start_idx(1, 1)
        start_gather(0)

        def pair(t, _):
            for b in (0, 1):
                j = 2 * t + b
                nb = 1 - b

                @pl.when(cid(j + 1) < NCHUNK)
                def _():
                    start_gather(nb)

                @pl.when(cid(j) < NCHUNK)
                def _():
                    pltpu.make_async_copy(table_hbm.at[idxb[b]], gb[b],
                                          sgb[b]).wait()

                @pl.when(cid(j + 2) < NCHUNK)
                def _():
                    start_idx(j + 2, b)

                @pl.when(jnp.logical_and(j >= 2, cid(j) < NCHUNK))
                def _():
                    pltpu.make_async_copy(
                        ob[b], out_hbm.at[pl.ds(0, CH)], sob[b]).wait()

                @pl.when(cid(j) < NCHUNK)
                def _():
                    reduce_store(j, b)

            return 0

        lax.fori_loop(0, NKMAX // 2, pair, 0)

        for b in (0, 1):
            pltpu.make_async_copy(ob[b], out_hbm.at[pl.ds(0, CH)],
                                  sob[b]).wait()

    return body


@functools.cache
def _build_calls():
    mesh = plsc.VectorSubcoreMesh(core_axis_name="c", subcore_axis_name="s")
    params = pltpu.CompilerParams(needs_layout_passes=False)
    encode = pl.kernel(
        _encode_body,
        out_type=jax.ShapeDtypeStruct((XS_SIZE,), jnp.float32),
        mesh=mesh,
        compiler_params=params,
        scratch_types=[
            pltpu.VMEM((BC,), jnp.float32),
            pltpu.VMEM((2 * BC,), jnp.float32),
            pltpu.VMEM((L,), jnp.float32),
            pltpu.SemaphoreType.DMA,
        ],
    )
    gather_scratch = [
        pltpu.VMEM((CE,), jnp.int32),
        pltpu.VMEM((CE,), jnp.int32),
        pltpu.VMEM((CE,), jnp.float32),
        pltpu.VMEM((CE,), jnp.float32),
        pltpu.VMEM((CH,), jnp.float32),
        pltpu.VMEM((CH,), jnp.float32),
        pltpu.SemaphoreType.DMA,
        pltpu.SemaphoreType.DMA,
        pltpu.SemaphoreType.DMA,
        pltpu.SemaphoreType.DMA,
        pltpu.SemaphoreType.DMA,
        pltpu.SemaphoreType.DMA,
    ]
    layer0 = pl.kernel(
        _make_gather_body(SHIFT, True),
        out_type=jax.ShapeDtypeStruct((OUT,), jnp.float32),
        mesh=mesh,
        compiler_params=params,
        scratch_types=gather_scratch,
    )
    layer1 = pl.kernel(
        _make_gather_body(0, False),
        out_type=jax.ShapeDtypeStruct((OUT,), jnp.float32),
        mesh=mesh,
        compiler_params=params,
        scratch_types=gather_scratch,
    )
    return encode, layer0, layer1


def kernel(weights, ptrs0, ptrs1):
    encode, layer0, layer1 = _build_calls()
    w_pad = jnp.concatenate(
        [weights, jnp.zeros((W_PAD - N_VARS,), jnp.float32)])
    xs = encode(w_pad)
    y0 = layer0(xs, ptrs0)
    y1 = layer1(y0, ptrs1)
    return y1
